# baseline (device time: 25765 ns/iter reference)
import jax
import jax.numpy as jnp
from jax import lax
from jax.experimental import pallas as pl
from jax.experimental.pallas import tpu as pltpu

N_DEV = 4
N_PEERS = N_DEV - 1


def kernel(x, Wq, K_ext, V_ext, Wo):
    b_loc, sq, d_model = x.shape
    _, hpb_x_dh = Wq.shape
    _, skv, hq, dh = K_ext.shape
    hpb = hpb_x_dh // dh
    rows = b_loc * sq

    my = lax.axis_index("i")
    Kb = lax.dynamic_slice_in_dim(K_ext, my * b_loc, b_loc, axis=0)
    Vb = lax.dynamic_slice_in_dim(V_ext, my * b_loc, b_loc, axis=0)
    Kb = jnp.reshape(Kb, (b_loc, skv, hq * dh))
    Vb = jnp.reshape(Vb, (b_loc, skv, hq * dh))
    Wq = Wq.astype(jnp.bfloat16)
    Wo = Wo.astype(jnp.bfloat16)

    def body(x_ref, wq_ref, k_scr, v_scr, wo_ref, out_ref,
             wq_com, wo_com, ctx_scr,
             wq_send, wq_recv, wo_send, wo_recv):
        my_pos = lax.axis_index("i")
        wq16 = wq_ref
        wo16 = wo_ref

        barrier_sem = pltpu.get_barrier_semaphore()
        for d in (1, 2, 3):
            pl.semaphore_signal(
                barrier_sem, inc=1,
                device_id=(lax.rem(my_pos + d, N_DEV),),
                device_id_type=pl.DeviceIdType.MESH,
            )
        pl.semaphore_wait(barrier_sem, N_PEERS)

        def make(src, dst, ssem, rsem, dev):
            return pltpu.make_async_remote_copy(
                src_ref=src, dst_ref=dst, send_sem=ssem, recv_sem=rsem,
                device_id=(dev,), device_id_type=pl.DeviceIdType.MESH,
            )

        sends = []
        for ref16, ssems, rsems, com in (
            (wq16, wq_send, wq_recv, wq_com),
            (wo16, wo_send, wo_recv, wo_com),
        ):
            for d in (2, 1, 3):
                s = 3 - d
                r = make(ref16, com.at[s], ssems.at[d - 1], rsems.at[s],
                         lax.rem(my_pos + d, N_DEV))
                r.start()
                sends.append(r)

        x16 = jnp.reshape(x_ref[...], (rows, d_model)).astype(jnp.bfloat16)

        ri = lax.broadcasted_iota(jnp.int32, (sq, skv), 0)
        ci = lax.broadcasted_iota(jnp.int32, (sq, skv), 1)
        qb = ri // 64
        kb = ci // 64
        mask = (qb == kb) | (kb == 0) | (((qb + kb) % 3) == 0)

        def attn_block(wq_blk, origin):
            q = jnp.dot(x16, wq_blk, preferred_element_type=jnp.float32)
            q = q.astype(jnp.bfloat16)
            for b in range(b_loc):
                q_b = q[b * sq:(b + 1) * sq, :]
                for p in range(hpb // 2):
                    pidx = origin * (hpb // 2) + p
                    slab_k = k_scr[b, :, pl.ds(pidx * 128, 128)].astype(
                        jnp.bfloat16)
                    slab_v = v_scr[b, :, pl.ds(pidx * 128, 128)].astype(
                        jnp.bfloat16)
                    for t in range(2):
                        hh = 2 * p + t
                        qh = q_b[:, hh * dh:(hh + 1) * dh]
                        kh = slab_k[:, t * dh:(t + 1) * dh]
                        vh = slab_v[:, t * dh:(t + 1) * dh]
                        s = lax.dot_general(
                            qh, kh, (((1,), (1,)), ((), ())),
                            preferred_element_type=jnp.float32,
                        ) * 0.125
                        s = jnp.where(mask, s, -1e9)
                        m = jnp.max(s, axis=-1, keepdims=True)
                        w = jnp.exp(s - m)
                        w = (w / jnp.sum(w, axis=-1, keepdims=True)
                             ).astype(jnp.bfloat16)
                        ch = jnp.dot(w, vh,
                                     preferred_element_type=jnp.float32)
                        ctx_scr[b * sq:(b + 1) * sq,
                                hh * dh:(hh + 1) * dh] = (
                            ch.astype(jnp.bfloat16))

        def out_block(wo_blk, acc):
            contrib = jnp.dot(ctx_scr[...], wo_blk,
                              preferred_element_type=jnp.float32)
            return contrib if acc is None else acc + contrib

        attn_block(wq16[...], my_pos)
        acc = out_block(wo16[...], None)

        for s in (0, 2, 1):
            make(wq16, wq_com.at[s], wq_send.at[s], wq_recv.at[s],
                 my_pos).wait_recv()
            attn_block(wq_com[s], lax.rem(my_pos + s + 1, N_DEV))
            make(wo16, wo_com.at[s], wo_send.at[s], wo_recv.at[s],
                 my_pos).wait_recv()
            acc = out_block(wo_com[s], acc)

        for r in sends:
            r.wait_send()

        for b in range(b_loc):
            out_ref[b, :, :] = acc[b * sq:(b + 1) * sq, :]

    return pl.pallas_call(
        body,
        out_shape=jax.ShapeDtypeStruct((b_loc, sq, d_model), jnp.float32),
        in_specs=[pl.BlockSpec(memory_space=pltpu.VMEM)] * 5,
        out_specs=pl.BlockSpec(memory_space=pltpu.VMEM),
        scratch_shapes=[
            pltpu.VMEM((N_PEERS, d_model, hpb * dh), jnp.bfloat16),
            pltpu.VMEM((N_PEERS, hpb * dh, d_model), jnp.bfloat16),
            pltpu.VMEM((rows, hpb * dh), jnp.bfloat16),
            pltpu.SemaphoreType.DMA((N_PEERS,)),
            pltpu.SemaphoreType.DMA((N_PEERS,)),
            pltpu.SemaphoreType.DMA((N_PEERS,)),
            pltpu.SemaphoreType.DMA((N_PEERS,)),
        ],
        compiler_params=pltpu.CompilerParams(collective_id=0),
    )(x, Wq, Kb, Vb, Wo)


# device time: 22976 ns/iter; 1.1214x vs baseline; 1.1214x over previous
import jax
import jax.numpy as jnp
from jax import lax
from jax.experimental import pallas as pl
from jax.experimental.pallas import tpu as pltpu

N_DEV = 4
N_PEERS = N_DEV - 1


def kernel(x, Wq, K_ext, V_ext, Wo):
    b_loc, sq, d_model = x.shape
    _, hpb_x_dh = Wq.shape
    _, skv, hq, dh = K_ext.shape
    hpb = hpb_x_dh // dh
    rows = b_loc * sq

    my = lax.axis_index("i")
    Kb = lax.dynamic_slice_in_dim(K_ext, my * b_loc, b_loc, axis=0)
    Vb = lax.dynamic_slice_in_dim(V_ext, my * b_loc, b_loc, axis=0)
    Kb = jnp.reshape(Kb, (b_loc, skv, hq * dh))
    Vb = jnp.reshape(Vb, (b_loc, skv, hq * dh))
    Wq = Wq.astype(jnp.bfloat16)
    Wo = Wo.astype(jnp.bfloat16)

    def body(x_ref, wq_ref, k_scr, v_scr, wo_ref, out_ref,
             wq_com, wo_com, ctx_scr,
             wq_send, wq_recv, wo_send, wo_recv):
        my_pos = lax.axis_index("i")
        wq16 = wq_ref
        wo16 = wo_ref

        barrier_sem = pltpu.get_barrier_semaphore()
        for d in (1, 2, 3):
            pl.semaphore_signal(
                barrier_sem, inc=1,
                device_id=(lax.rem(my_pos + d, N_DEV),),
                device_id_type=pl.DeviceIdType.MESH,
            )
        pl.semaphore_wait(barrier_sem, N_PEERS)

        def make(src, dst, ssem, rsem, dev):
            return pltpu.make_async_remote_copy(
                src_ref=src, dst_ref=dst, send_sem=ssem, recv_sem=rsem,
                device_id=(dev,), device_id_type=pl.DeviceIdType.MESH,
            )

        sends = []
        for ref16, ssems, rsems, com in (
            (wq16, wq_send, wq_recv, wq_com),
            (wo16, wo_send, wo_recv, wo_com),
        ):
            for d in (1, 3, 2):
                s = 3 - d
                r = make(ref16, com.at[s], ssems.at[d - 1], rsems.at[s],
                         lax.rem(my_pos + d, N_DEV))
                r.start()
                sends.append(r)

        x16 = jnp.reshape(x_ref[...], (rows, d_model)).astype(jnp.bfloat16)

        ri = lax.broadcasted_iota(jnp.int32, (sq, skv), 0)
        ci = lax.broadcasted_iota(jnp.int32, (sq, skv), 1)
        qb = ri // 64
        kb = ci // 64
        mask = (qb == kb) | (kb == 0) | (((qb + kb) % 3) == 0)

        def attn_block(wq_blk, origin):
            q = jnp.dot(x16, wq_blk, preferred_element_type=jnp.float32)
            q = q.astype(jnp.bfloat16)
            for b in range(b_loc):
                q_b = q[b * sq:(b + 1) * sq, :]
                for p in range(hpb // 2):
                    pidx = origin * (hpb // 2) + p
                    slab_k = k_scr[b, :, pl.ds(pidx * 128, 128)].astype(
                        jnp.bfloat16)
                    slab_v = v_scr[b, :, pl.ds(pidx * 128, 128)].astype(
                        jnp.bfloat16)
                    for t in range(2):
                        hh = 2 * p + t
                        qh = q_b[:, hh * dh:(hh + 1) * dh]
                        kh = slab_k[:, t * dh:(t + 1) * dh]
                        vh = slab_v[:, t * dh:(t + 1) * dh]
                        s = lax.dot_general(
                            qh, kh, (((1,), (1,)), ((), ())),
                            preferred_element_type=jnp.float32,
                        ) * 0.125
                        s = jnp.where(mask, s, -1e9)
                        m = jnp.max(s, axis=-1, keepdims=True)
                        w = jnp.exp(s - m)
                        w = (w / jnp.sum(w, axis=-1, keepdims=True)
                             ).astype(jnp.bfloat16)
                        ch = jnp.dot(w, vh,
                                     preferred_element_type=jnp.float32)
                        ctx_scr[b * sq:(b + 1) * sq,
                                hh * dh:(hh + 1) * dh] = (
                            ch.astype(jnp.bfloat16))

        def out_block(wo_blk, acc):
            contrib = jnp.dot(ctx_scr[...], wo_blk,
                              preferred_element_type=jnp.float32)
            return contrib if acc is None else acc + contrib

        attn_block(wq16[...], my_pos)
        acc = out_block(wo16[...], None)

        for s in (0, 2, 1):
            make(wq16, wq_com.at[s], wq_send.at[s], wq_recv.at[s],
                 my_pos).wait_recv()
            attn_block(wq_com[s], lax.rem(my_pos + s + 1, N_DEV))
            make(wo16, wo_com.at[s], wo_send.at[s], wo_recv.at[s],
                 my_pos).wait_recv()
            acc = out_block(wo_com[s], acc)

        for r in sends:
            r.wait_send()

        for b in range(b_loc):
            out_ref[b, :, :] = acc[b * sq:(b + 1) * sq, :]

    return pl.pallas_call(
        body,
        out_shape=jax.ShapeDtypeStruct((b_loc, sq, d_model), jnp.float32),
        in_specs=[pl.BlockSpec(memory_space=pltpu.VMEM)] * 5,
        out_specs=pl.BlockSpec(memory_space=pltpu.VMEM),
        scratch_shapes=[
            pltpu.VMEM((N_PEERS, d_model, hpb * dh), jnp.bfloat16),
            pltpu.VMEM((N_PEERS, hpb * dh, d_model), jnp.bfloat16),
            pltpu.VMEM((rows, hpb * dh), jnp.bfloat16),
            pltpu.SemaphoreType.DMA((N_PEERS,)),
            pltpu.SemaphoreType.DMA((N_PEERS,)),
            pltpu.SemaphoreType.DMA((N_PEERS,)),
            pltpu.SemaphoreType.DMA((N_PEERS,)),
        ],
        compiler_params=pltpu.CompilerParams(collective_id=0),
    )(x, Wq, Kb, Vb, Wo)
